# baseline (speedup 1.0000x reference)
import math

import jax
import jax.numpy as jnp
from jax.experimental import pallas as pl

_TEMP_INV = 1.0 / math.e
_N_EXPERTS = 8
_N_ACTIVE = 5


def _gate_body(x_ref, w_ref, b_ref, o_ref):
    x = x_ref[...]
    w = w_ref[...]
    s = jax.lax.dot_general(
        w, x, (((1,), (1,)), ((), ())),
        preferred_element_type=jnp.float32)
    s = (s + b_ref[...]) * _TEMP_INV
    m = jnp.max(s, axis=0, keepdims=True)
    e = jnp.exp(s - m)
    z = jnp.sum(e, axis=0, keepdims=True)
    p = e / z

    rows = []
    for i in range(_N_EXPERTS):
        pi = p[i:i + 1, :]
        gt = (p > pi).astype(jnp.float32)
        tie = (p[:i] == pi).astype(jnp.float32) if i else None
        rank = jnp.sum(gt, axis=0, keepdims=True)
        if tie is not None:
            rank = rank + jnp.sum(tie, axis=0, keepdims=True)
        rows.append((rank < _N_ACTIVE).astype(jnp.float32))
    keep = jnp.concatenate(rows, axis=0)

    kept = p * keep
    denom = jnp.sum(kept, axis=0, keepdims=True) + 1e-8
    o_ref[...] = kept / denom


def kernel(x, W, b):
    n, d = x.shape
    rows = 8192
    grid = (n // rows,)
    b2 = b.reshape(_N_EXPERTS, 1)
    out_t = pl.pallas_call(
        _gate_body,
        grid=grid,
        in_specs=[
            pl.BlockSpec((rows, d), lambda i: (i, 0)),
            pl.BlockSpec((_N_EXPERTS, d), lambda i: (0, 0)),
            pl.BlockSpec((_N_EXPERTS, 1), lambda i: (0, 0)),
        ],
        out_specs=pl.BlockSpec((_N_EXPERTS, rows), lambda i: (0, i)),
        out_shape=jax.ShapeDtypeStruct((_N_EXPERTS, n), jnp.float32),
    )(x, W, b2)
    return out_t.T
